# baseline (speedup 1.0000x reference)
import functools

import jax
import jax.numpy as jnp
from jax import lax
from jax.experimental import pallas as pl
from jax.experimental.pallas import tpu as pltpu
from jax.experimental.pallas import tpu_sc as plsc

NC = 2
NS = 16
NW = NC * NS
L = 16


def _sc_edge_agg(n_nodes, d, n_edges, chunk, zrows, dn=None):
    e_per_w = n_edges // NW
    n_chunks = e_per_w // chunk
    rows_per_tile = (n_nodes // NS) // 8 * 8
    tail = n_nodes - rows_per_tile * NS
    n_zcopy = rows_per_tile // zrows
    da = dn if dn is not None else d
    assert e_per_w * NW == n_edges and n_chunks * chunk == e_per_w
    assert n_zcopy * zrows == rows_per_tile and 0 <= tail <= zrows and tail % 8 == 0
    assert chunk % 8 == 0 and chunk <= 128 and d % L == 0 and da % L == 0

    mesh = plsc.VectorSubcoreMesh(core_axis_name="c", subcore_axis_name="s")

    scratch = [
        pltpu.VMEM((2, chunk), jnp.int32),
        pltpu.VMEM((2, chunk), jnp.int32),
        pltpu.VMEM((2, chunk, d), jnp.float32),
        pltpu.VMEM((zrows, da), jnp.float32),
        pltpu.VMEM_SHARED((n_nodes, da), jnp.float32),
        pltpu.SemaphoreType.DMA,
        pltpu.SemaphoreType.DMA,
        pltpu.SemaphoreType.DMA,
        pltpu.SemaphoreType.DMA,
    ]
    if dn is not None:
        scratch.append(pltpu.VMEM((2, chunk, dn), jnp.float32))

    @functools.partial(
        pl.kernel,
        mesh=mesh,
        out_type=jax.ShapeDtypeStruct((NC, n_nodes, da), jnp.float32),
        scratch_types=scratch,
    )
    def agg(src_hbm, dst_hbm, vals_hbm, out_hbm,
            sbuf, dbuf, rows_v, zero_v, acc_sh, sg0, sg1, si0, si1,
            *maybe_rows_n):
        cid = lax.axis_index("c")
        sid = lax.axis_index("s")
        wid = cid * NS + sid
        row0 = sid * rows_per_tile

        zvec = jnp.zeros((L,), jnp.float32)
        dl = da // L

        def zstore(i, carry):
            zero_v[i // dl, pl.ds((i % dl) * L, L)] = zvec
            return carry

        lax.fori_loop(0, zrows * dl, zstore, 0)

        def zcopy(j, carry):
            pltpu.sync_copy(zero_v, acc_sh.at[pl.ds(row0 + j * zrows, zrows)])
            return carry

        lax.fori_loop(0, n_zcopy, zcopy, 0)

        if tail:
            @pl.when(sid == NS - 1)
            def _():
                pltpu.sync_copy(zero_v.at[pl.ds(0, tail)],
                                acc_sh.at[pl.ds(n_nodes - tail, tail)])

        gather_src = vals_hbm
        plsc.subcore_barrier()

        e_base = wid * e_per_w
        sg = (sg0, sg1)
        si = (si0, si1)

        def idx_start(i, b):
            e0 = e_base + i * chunk
            pltpu.async_copy(src_hbm.at[pl.ds(e0, chunk)], sbuf.at[b], si[b])
            pltpu.async_copy(dst_hbm.at[pl.ds(e0, chunk)], dbuf.at[b], si[b])

        def idx_wait(b):
            pltpu.make_async_copy(src_hbm.at[pl.ds(0, chunk)], sbuf.at[b], si[b]).wait()
            pltpu.make_async_copy(dst_hbm.at[pl.ds(0, chunk)], dbuf.at[b], si[b]).wait()

        def g_start(b):
            pltpu.async_copy(gather_src.at[sbuf.at[b]], rows_v.at[b], sg[b])

        def g_wait(b):
            pltpu.make_async_copy(gather_src.at[sbuf.at[b]], rows_v.at[b], sg[b]).wait()

        idx_start(0, 0)
        idx_start(1, 1)
        idx_wait(0)
        g_start(0)

        def pair(k, carry):
            for b in (0, 1):
                i = 2 * k + b

                @pl.when(i < n_chunks)
                def _():
                    @pl.when(i + 1 < n_chunks)
                    def _():
                        idx_wait(1 - b)

                    g_wait(b)

                    @pl.when(i + 1 < n_chunks)
                    def _():
                        g_start(1 - b)

                    if dn is None:
                        pltpu.sync_copy(rows_v.at[b], acc_sh.at[dbuf.at[b]],
                                        add=True)
                    else:
                        rows_n = maybe_rows_n[0]

                        def ebody(j, carry):
                            for c in range(dn // L):
                                rows_n[b, j, pl.ds(c * L, L)] = (
                                    rows_v[b, j, pl.ds(c * L, L)])
                            return carry

                        lax.fori_loop(0, chunk, ebody, 0)
                        pltpu.sync_copy(rows_n.at[b], acc_sh.at[dbuf.at[b]],
                                        add=True)

                    @pl.when(i + 2 < n_chunks)
                    def _():
                        idx_start(i + 2, b)
            return carry

        lax.fori_loop(0, (n_chunks + 1) // 2, pair, 0)
        plsc.subcore_barrier()

        pltpu.sync_copy(acc_sh.at[pl.ds(row0, rows_per_tile)],
                        out_hbm.at[cid, pl.ds(row0, rows_per_tile)])

        if tail:
            @pl.when(sid == NS - 1)
            def _():
                pltpu.sync_copy(acc_sh.at[pl.ds(n_nodes - tail, tail)],
                                out_hbm.at[cid, pl.ds(n_nodes - tail, tail)])

    return agg


def _sc_edge_agg_packed(n_nodes, n_edges, chunk):
    d = 128
    T = n_nodes * 8
    e_per_w = n_edges // NW
    n_chunks = e_per_w // chunk
    share = 4992
    tail = T - share * NS
    assert share % 128 == 0 and tail % 128 == 0 and (T // 16) * 16 == T
    assert e_per_w * NW == n_edges and n_chunks * chunk == e_per_w
    assert chunk % 8 == 0 and chunk <= 128

    mesh = plsc.VectorSubcoreMesh(core_axis_name="c", subcore_axis_name="s")

    scratch = [
        pltpu.VMEM((2, chunk), jnp.int32),
        pltpu.VMEM((2, chunk), jnp.int32),
        pltpu.VMEM((2, chunk, d), jnp.float32),
        pltpu.VMEM((T,), jnp.float32),
        pltpu.SemaphoreType.DMA,
        pltpu.SemaphoreType.DMA,
        pltpu.SemaphoreType.DMA,
        pltpu.SemaphoreType.DMA,
    ]

    @functools.partial(
        pl.kernel,
        mesh=mesh,
        out_type=jax.ShapeDtypeStruct((NW * T,), jnp.float32),
        scratch_types=scratch,
        compiler_params=pltpu.CompilerParams(needs_layout_passes=False),
    )
    def agg(src_hbm, dst_hbm, vals_hbm, out_hbm,
            sbuf, dbuf, rows_v, tbl, sg0, sg1, si0, si1):
        cid = lax.axis_index("c")
        sid = lax.axis_index("s")
        wid = cid * NS + sid

        zvec = jnp.zeros((L,), jnp.float32)

        def ztbl(i, carry):
            tbl[pl.ds(i * L, L)] = zvec
            return carry

        lax.fori_loop(0, T // L, ztbl, 0)

        e_base = wid * e_per_w
        sg = (sg0, sg1)
        si = (si0, si1)
        lane = lax.iota(jnp.int32, L)
        lmask = lane < 8

        def bcast(vec, u):
            idx = jnp.full((L,), u, dtype=jnp.int32)
            return lax.gather(
                vec, idx[:, None],
                dimension_numbers=lax.GatherDimensionNumbers(
                    offset_dims=(), collapsed_slice_dims=(0,),
                    start_index_map=(0,)),
                slice_sizes=(1,),
                mode=lax.GatherScatterMode.PROMISE_IN_BOUNDS)

        def idx_start(i, b):
            e0 = e_base + i * chunk
            pltpu.async_copy(src_hbm.at[pl.ds(e0, chunk)], sbuf.at[b], si[b])
            pltpu.async_copy(dst_hbm.at[pl.ds(e0, chunk)], dbuf.at[b], si[b])

        def idx_wait(b):
            pltpu.make_async_copy(src_hbm.at[pl.ds(0, chunk)], sbuf.at[b], si[b]).wait()
            pltpu.make_async_copy(dst_hbm.at[pl.ds(0, chunk)], dbuf.at[b], si[b]).wait()

        def g_start(b):
            pltpu.async_copy(vals_hbm.at[sbuf.at[b]], rows_v.at[b], sg[b])

        def g_wait(b):
            pltpu.make_async_copy(vals_hbm.at[sbuf.at[b]], rows_v.at[b], sg[b]).wait()

        idx_start(0, 0)
        idx_start(1, 1)
        idx_wait(0)
        g_start(0)

        def pair(k, carry):
            for b in (0, 1):
                i = 2 * k + b

                @pl.when(i < n_chunks)
                def _():
                    @pl.when(i + 1 < n_chunks)
                    def _():
                        idx_wait(1 - b)

                    g_wait(b)

                    @pl.when(i + 1 < n_chunks)
                    def _():
                        g_start(1 - b)

                    def ebody(g, carry2):
                        dv8 = dbuf[b, pl.ds(g * L, L)] * 8
                        for u in range(L):
                            addr = bcast(dv8, u) + lane
                            plsc.addupdate_scatter(
                                tbl, [addr], rows_v[b, g * L + u, pl.ds(0, L)],
                                mask=lmask)
                        return carry2

                    lax.fori_loop(0, chunk // L, ebody, 0)

                    @pl.when(i + 2 < n_chunks)
                    def _():
                        idx_start(i + 2, b)
            return carry

        lax.fori_loop(0, (n_chunks + 1) // 2, pair, 0)

        pltpu.sync_copy(tbl, out_hbm.at[pl.ds(wid * T, T)])

    return agg


def _mid_body(p0, p1, w1, b1, w2, out):
    agg = p0[0] + p1[0]
    h = jnp.maximum(
        jnp.dot(agg, w1[...], preferred_element_type=jnp.float32) + b1[...], 0.0)
    out[...] = jnp.dot(h, w2[...], preferred_element_type=jnp.float32)


def _fin_body(qref, b2, out):
    out[...] = jnp.sum(qref[...], axis=0) + b2[...]


def kernel(features, edge_index, W1, b1, W2, b2):
    n, d = features.shape
    e = edge_index.shape[1]
    d2 = 128
    dn = 16
    block = 1000

    src = edge_index[0]
    dst = edge_index[1]
    W2p = jnp.pad(W2, ((0, 0), (0, d2 - W2.shape[1])))
    b2p = jnp.pad(b2, (0, d2 - b2.shape[0])).reshape(1, d2)
    b1r = b1.reshape(1, d)

    p = _sc_edge_agg(n, d, e, 80, 48)(src, dst, features)

    hw2 = pl.pallas_call(
        _mid_body,
        grid=(n // block,),
        in_specs=[
            pl.BlockSpec((1, block, d), lambda i: (0, i, 0)),
            pl.BlockSpec((1, block, d), lambda i: (1, i, 0)),
            pl.BlockSpec((d, d), lambda i: (0, 0)),
            pl.BlockSpec((1, d), lambda i: (0, 0)),
            pl.BlockSpec((d, d2), lambda i: (0, 0)),
        ],
        out_specs=pl.BlockSpec((block, d2), lambda i: (i, 0)),
        out_shape=jax.ShapeDtypeStruct((n, d2), jnp.float32),
    )(p, p, W1, b1r, W2p)

    q = _sc_edge_agg_packed(n, e, 80)(src, dst, hw2)
    npack = n * 8 // d2
    qr = q.reshape(NW, npack, d2)

    b2t = jnp.tile(jnp.pad(b2, (0, 8 - b2.shape[0])), d2 // 8).reshape(1, d2)
    rblk = npack
    packed = pl.pallas_call(
        _fin_body,
        grid=(npack // rblk,),
        in_specs=[
            pl.BlockSpec((NW, rblk, d2), lambda i: (0, i, 0)),
            pl.BlockSpec((1, d2), lambda i: (0, 0)),
        ],
        out_specs=pl.BlockSpec((rblk, d2), lambda i: (i, 0)),
        out_shape=jax.ShapeDtypeStruct((npack, d2), jnp.float32),
    )(qr, b2t)

    return lax.slice(packed.reshape(n, 8), (0, 0), (n, 7))


# speedup vs baseline: 1.1949x; 1.1949x over previous
import functools

import jax
import jax.numpy as jnp
from jax import lax
from jax.experimental import pallas as pl
from jax.experimental.pallas import tpu as pltpu
from jax.experimental.pallas import tpu_sc as plsc

NC = 2
NS = 16
NW = NC * NS
L = 16


def _sc_edge_agg(n_nodes, d, n_edges, chunk, zrows, dn=None):
    e_per_w = n_edges // NW
    n_chunks = e_per_w // chunk
    rows_per_tile = (n_nodes // NS) // 8 * 8
    tail = n_nodes - rows_per_tile * NS
    n_zcopy = rows_per_tile // zrows
    da = dn if dn is not None else d
    assert e_per_w * NW == n_edges and n_chunks * chunk == e_per_w
    assert n_zcopy * zrows == rows_per_tile and 0 <= tail <= zrows and tail % 8 == 0
    assert chunk % 8 == 0 and chunk <= 128 and d % L == 0 and da % L == 0

    mesh = plsc.VectorSubcoreMesh(core_axis_name="c", subcore_axis_name="s")

    scratch = [
        pltpu.VMEM((3, chunk), jnp.int32),
        pltpu.VMEM((3, chunk), jnp.int32),
        pltpu.VMEM((3, chunk, d), jnp.float32),
        pltpu.VMEM((zrows, da), jnp.float32),
        pltpu.VMEM_SHARED((n_nodes, da), jnp.float32),
        pltpu.SemaphoreType.DMA,
        pltpu.SemaphoreType.DMA,
        pltpu.SemaphoreType.DMA,
        pltpu.SemaphoreType.DMA,
        pltpu.SemaphoreType.DMA,
        pltpu.SemaphoreType.DMA,
    ]

    @functools.partial(
        pl.kernel,
        mesh=mesh,
        out_type=jax.ShapeDtypeStruct((NC, n_nodes, da), jnp.float32),
        scratch_types=scratch,
    )
    def agg(src_hbm, dst_hbm, vals_hbm, out_hbm,
            sbuf, dbuf, rows_v, zero_v, acc_sh,
            sg0, sg1, sg2, si0, si1, si2):
        cid = lax.axis_index("c")
        sid = lax.axis_index("s")
        wid = cid * NS + sid
        row0 = sid * rows_per_tile

        zvec = jnp.zeros((L,), jnp.float32)
        dl = da // L

        def zstore(i, carry):
            zero_v[i // dl, pl.ds((i % dl) * L, L)] = zvec
            return carry

        lax.fori_loop(0, zrows * dl, zstore, 0)

        def zcopy(j, carry):
            pltpu.sync_copy(zero_v, acc_sh.at[pl.ds(row0 + j * zrows, zrows)])
            return carry

        lax.fori_loop(0, n_zcopy, zcopy, 0)

        if tail:
            @pl.when(sid == NS - 1)
            def _():
                pltpu.sync_copy(zero_v.at[pl.ds(0, tail)],
                                acc_sh.at[pl.ds(n_nodes - tail, tail)])

        gather_src = vals_hbm
        plsc.subcore_barrier()

        e_base = wid * e_per_w
        sg = (sg0, sg1, sg2)
        si = (si0, si1, si2)

        def idx_start(i, b):
            e0 = e_base + i * chunk
            pltpu.async_copy(src_hbm.at[pl.ds(e0, chunk)], sbuf.at[b], si[b])
            pltpu.async_copy(dst_hbm.at[pl.ds(e0, chunk)], dbuf.at[b], si[b])

        def idx_wait(b):
            pltpu.make_async_copy(src_hbm.at[pl.ds(0, chunk)], sbuf.at[b], si[b]).wait()
            pltpu.make_async_copy(dst_hbm.at[pl.ds(0, chunk)], dbuf.at[b], si[b]).wait()

        def g_start(b):
            pltpu.async_copy(gather_src.at[sbuf.at[b]], rows_v.at[b], sg[b])

        def g_wait(b):
            pltpu.make_async_copy(gather_src.at[sbuf.at[b]], rows_v.at[b], sg[b]).wait()

        idx_start(0, 0)
        idx_start(1, 1)
        idx_start(2, 2)
        idx_wait(0)
        g_start(0)
        idx_wait(1)
        g_start(1)

        def trip(k, carry):
            for b in (0, 1, 2):
                i = 3 * k + b
                b2i = (b + 2) % 3

                @pl.when(i < n_chunks)
                def _():
                    g_wait(b)

                    @pl.when(i + 2 < n_chunks)
                    def _():
                        idx_wait(b2i)
                        g_start(b2i)

                    pltpu.sync_copy(rows_v.at[b], acc_sh.at[dbuf.at[b]],
                                    add=True)

                    @pl.when(i + 3 < n_chunks)
                    def _():
                        idx_start(i + 3, b)
            return carry

        lax.fori_loop(0, (n_chunks + 2) // 3, trip, 0)
        plsc.subcore_barrier()

        pltpu.sync_copy(acc_sh.at[pl.ds(row0, rows_per_tile)],
                        out_hbm.at[cid, pl.ds(row0, rows_per_tile)])

        if tail:
            @pl.when(sid == NS - 1)
            def _():
                pltpu.sync_copy(acc_sh.at[pl.ds(n_nodes - tail, tail)],
                                out_hbm.at[cid, pl.ds(n_nodes - tail, tail)])

    return agg


def _sc_edge_agg_packed(n_nodes, n_edges, chunk):
    d = 128
    T = n_nodes * 8
    e_per_w = n_edges // NW
    n_chunks = e_per_w // chunk
    share = 4992
    tail = T - share * NS
    assert share % 128 == 0 and tail % 128 == 0 and (T // 16) * 16 == T
    assert e_per_w * NW == n_edges and n_chunks * chunk == e_per_w
    assert chunk % 8 == 0 and chunk <= 128

    mesh = plsc.VectorSubcoreMesh(core_axis_name="c", subcore_axis_name="s")

    scratch = [
        pltpu.VMEM((2, chunk), jnp.int32),
        pltpu.VMEM((2, chunk), jnp.int32),
        pltpu.VMEM((2, chunk, d), jnp.float32),
        pltpu.VMEM((T,), jnp.float32),
        pltpu.SemaphoreType.DMA,
        pltpu.SemaphoreType.DMA,
        pltpu.SemaphoreType.DMA,
        pltpu.SemaphoreType.DMA,
    ]

    @functools.partial(
        pl.kernel,
        mesh=mesh,
        out_type=jax.ShapeDtypeStruct((NW * T,), jnp.float32),
        scratch_types=scratch,
        compiler_params=pltpu.CompilerParams(needs_layout_passes=False),
    )
    def agg(src_hbm, dst_hbm, vals_hbm, out_hbm,
            sbuf, dbuf, rows_v, tbl, sg0, sg1, si0, si1):
        cid = lax.axis_index("c")
        sid = lax.axis_index("s")
        wid = cid * NS + sid

        zvec = jnp.zeros((L,), jnp.float32)

        def ztbl(i, carry):
            tbl[pl.ds(i * L, L)] = zvec
            return carry

        lax.fori_loop(0, T // L, ztbl, 0)

        e_base = wid * e_per_w
        sg = (sg0, sg1, sg2)
        si = (si0, si1, si2)
        lane = lax.iota(jnp.int32, L)
        lmask = lane < 8

        def bcast(vec, u):
            idx = jnp.full((L,), u, dtype=jnp.int32)
            return lax.gather(
                vec, idx[:, None],
                dimension_numbers=lax.GatherDimensionNumbers(
                    offset_dims=(), collapsed_slice_dims=(0,),
                    start_index_map=(0,)),
                slice_sizes=(1,),
                mode=lax.GatherScatterMode.PROMISE_IN_BOUNDS)

        def idx_start(i, b):
            e0 = e_base + i * chunk
            pltpu.async_copy(src_hbm.at[pl.ds(e0, chunk)], sbuf.at[b], si[b])
            pltpu.async_copy(dst_hbm.at[pl.ds(e0, chunk)], dbuf.at[b], si[b])

        def idx_wait(b):
            pltpu.make_async_copy(src_hbm.at[pl.ds(0, chunk)], sbuf.at[b], si[b]).wait()
            pltpu.make_async_copy(dst_hbm.at[pl.ds(0, chunk)], dbuf.at[b], si[b]).wait()

        def g_start(b):
            pltpu.async_copy(vals_hbm.at[sbuf.at[b]], rows_v.at[b], sg[b])

        def g_wait(b):
            pltpu.make_async_copy(vals_hbm.at[sbuf.at[b]], rows_v.at[b], sg[b]).wait()

        idx_start(0, 0)
        idx_start(1, 1)
        idx_wait(0)
        g_start(0)

        def pair(k, carry):
            for b in (0, 1):
                i = 2 * k + b

                @pl.when(i < n_chunks)
                def _():
                    @pl.when(i + 1 < n_chunks)
                    def _():
                        idx_wait(1 - b)

                    g_wait(b)

                    @pl.when(i + 1 < n_chunks)
                    def _():
                        g_start(1 - b)

                    def ebody(g, carry2):
                        dv8 = dbuf[b, pl.ds(g * L, L)] * 8
                        for u in range(L):
                            addr = bcast(dv8, u) + lane
                            plsc.addupdate_scatter(
                                tbl, [addr], rows_v[b, g * L + u, pl.ds(0, L)],
                                mask=lmask)
                        return carry2

                    lax.fori_loop(0, chunk // L, ebody, 0)

                    @pl.when(i + 2 < n_chunks)
                    def _():
                        idx_start(i + 2, b)
            return carry

        lax.fori_loop(0, (n_chunks + 1) // 2, pair, 0)

        pltpu.sync_copy(tbl, out_hbm.at[pl.ds(wid * T, T)])

    return agg


def _mid_body(p0, p1, w1, b1, w2, out):
    agg = p0[0] + p1[0]
    h = jnp.maximum(
        jnp.dot(agg, w1[...], preferred_element_type=jnp.float32) + b1[...], 0.0)
    out[...] = jnp.dot(h, w2[...], preferred_element_type=jnp.float32)


def _fin_body(q0, q1, b2, out):
    out[...] = q0[0] + q1[0] + b2[...]


def kernel(features, edge_index, W1, b1, W2, b2):
    n, d = features.shape
    e = edge_index.shape[1]
    d2 = 128
    dn = 16
    block = 1000

    src = edge_index[0]
    dst = edge_index[1]
    W2p = jnp.pad(W2, ((0, 0), (0, d2 - W2.shape[1])))
    b1r = b1.reshape(1, d)

    p = _sc_edge_agg(n, d, e, 80, 48)(src, dst, features)

    hw2 = pl.pallas_call(
        _mid_body,
        grid=(n // block,),
        in_specs=[
            pl.BlockSpec((1, block, d), lambda i: (0, i, 0)),
            pl.BlockSpec((1, block, d), lambda i: (1, i, 0)),
            pl.BlockSpec((d, d), lambda i: (0, 0)),
            pl.BlockSpec((1, d), lambda i: (0, 0)),
            pl.BlockSpec((d, d2), lambda i: (0, 0)),
        ],
        out_specs=pl.BlockSpec((block, d2), lambda i: (i, 0)),
        out_shape=jax.ShapeDtypeStruct((n, d2), jnp.float32),
    )(p, p, W1, b1r, W2p)

    q = _sc_edge_agg(n, d2, e, 80, 48)(src, dst, hw2)

    b2p = jnp.pad(b2, (0, d2 - b2.shape[0])).reshape(1, d2)
    logits128 = pl.pallas_call(
        _fin_body,
        grid=(n // block,),
        in_specs=[
            pl.BlockSpec((1, block, d2), lambda i: (0, i, 0)),
            pl.BlockSpec((1, block, d2), lambda i: (1, i, 0)),
            pl.BlockSpec((1, d2), lambda i: (0, 0)),
        ],
        out_specs=pl.BlockSpec((block, d2), lambda i: (i, 0)),
        out_shape=jax.ShapeDtypeStruct((n, d2), jnp.float32),
    )(q, q, b2p)

    return lax.slice(logits128, (0, 0), (n, 7))


# speedup vs baseline: 1.5314x; 1.2816x over previous
import functools

import jax
import jax.numpy as jnp
from jax import lax
from jax.experimental import pallas as pl
from jax.experimental.pallas import tpu as pltpu
from jax.experimental.pallas import tpu_sc as plsc

NC = 2
NS = 16
NW = NC * NS
L = 16


def _sc_edge_agg(n_nodes, d, n_edges, chunk, zrows, dn=None):
    e_per_w = n_edges // NW
    n_chunks = e_per_w // chunk
    rows_per_tile = (n_nodes // NS) // 8 * 8
    tail = n_nodes - rows_per_tile * NS
    n_zcopy = rows_per_tile // zrows
    da = dn if dn is not None else d
    assert e_per_w * NW == n_edges and n_chunks * chunk == e_per_w
    assert n_zcopy * zrows == rows_per_tile and 0 <= tail <= zrows and tail % 8 == 0
    assert chunk % 8 == 0 and chunk <= 128 and d % L == 0 and da % L == 0

    mesh = plsc.VectorSubcoreMesh(core_axis_name="c", subcore_axis_name="s")

    scratch = [
        pltpu.VMEM((3, chunk), jnp.int32),
        pltpu.VMEM((6, chunk), jnp.int32),
        pltpu.VMEM((3, chunk, d), jnp.float32),
        pltpu.VMEM((zrows, da), jnp.float32),
        pltpu.VMEM_SHARED((n_nodes, da), jnp.float32),
        pltpu.SemaphoreType.DMA,
        pltpu.SemaphoreType.DMA,
        pltpu.SemaphoreType.DMA,
        pltpu.SemaphoreType.DMA,
        pltpu.SemaphoreType.DMA,
        pltpu.SemaphoreType.DMA,
        pltpu.SemaphoreType.DMA,
        pltpu.SemaphoreType.DMA,
        pltpu.SemaphoreType.DMA,
    ]

    @functools.partial(
        pl.kernel,
        mesh=mesh,
        out_type=jax.ShapeDtypeStruct((NC, n_nodes, da), jnp.float32),
        scratch_types=scratch,
    )
    def agg(src_hbm, dst_hbm, vals_hbm, out_hbm,
            sbuf, dbuf, rows_v, zero_v, acc_sh,
            sg0, sg1, sg2, si0, si1, si2, sc0, sc1, sc2):
        cid = lax.axis_index("c")
        sid = lax.axis_index("s")
        wid = cid * NS + sid
        row0 = sid * rows_per_tile

        zvec = jnp.zeros((L,), jnp.float32)
        dl = da // L

        def zstore(i, carry):
            zero_v[i // dl, pl.ds((i % dl) * L, L)] = zvec
            return carry

        lax.fori_loop(0, zrows * dl, zstore, 0)

        def zcopy(j, carry):
            pltpu.sync_copy(zero_v, acc_sh.at[pl.ds(row0 + j * zrows, zrows)])
            return carry

        lax.fori_loop(0, n_zcopy, zcopy, 0)

        if tail:
            @pl.when(sid == NS - 1)
            def _():
                pltpu.sync_copy(zero_v.at[pl.ds(0, tail)],
                                acc_sh.at[pl.ds(n_nodes - tail, tail)])

        gather_src = vals_hbm
        plsc.subcore_barrier()

        e_base = wid * e_per_w
        sg = (sg0, sg1, sg2)
        si = (si0, si1, si2)
        sc = (sc0, sc1, sc2)

        def idx_start(i, b, b6):
            e0 = e_base + i * chunk
            pltpu.async_copy(src_hbm.at[pl.ds(e0, chunk)], sbuf.at[b], si[b])
            pltpu.async_copy(dst_hbm.at[pl.ds(e0, chunk)], dbuf.at[b6], si[b])

        def idx_wait(b, b6):
            pltpu.make_async_copy(src_hbm.at[pl.ds(0, chunk)], sbuf.at[b], si[b]).wait()
            pltpu.make_async_copy(dst_hbm.at[pl.ds(0, chunk)], dbuf.at[b6], si[b]).wait()

        def sc_start(b, b6):
            pltpu.async_copy(rows_v.at[b], acc_sh.at[dbuf.at[b6]], sc[b], add=True)

        def sc_wait(b, b6):
            pltpu.make_async_copy(rows_v.at[b], acc_sh.at[dbuf.at[b6]], sc[b]).wait()

        def g_start(b):
            pltpu.async_copy(gather_src.at[sbuf.at[b]], rows_v.at[b], sg[b])

        def g_wait(b):
            pltpu.make_async_copy(gather_src.at[sbuf.at[b]], rows_v.at[b], sg[b]).wait()

        idx_start(0, 0, 0)
        idx_start(1, 1, 1)
        idx_start(2, 2, 2)
        idx_wait(0, 0)
        g_start(0)
        idx_wait(1, 1)
        g_start(1)

        def hexa(k, carry):
            for b6 in range(6):
                i = 6 * k + b6
                b = b6 % 3
                b2i = (b + 2) % 3

                @pl.when(i < n_chunks)
                def _():
                    g_wait(b)

                    @pl.when(i >= 1)
                    def _():
                        sc_wait((b + 2) % 3, (b6 + 5) % 6)

                    sc_start(b, b6)

                    @pl.when(i + 2 < n_chunks)
                    def _():
                        idx_wait(b2i, (b6 + 2) % 6)
                        g_start(b2i)

                    @pl.when(i + 3 < n_chunks)
                    def _():
                        idx_start(i + 3, b, (b6 + 3) % 6)
            return carry

        lax.fori_loop(0, (n_chunks + 5) // 6, hexa, 0)
        sc_wait((n_chunks - 1) % 3, (n_chunks - 1) % 6)
        plsc.subcore_barrier()

        pltpu.sync_copy(acc_sh.at[pl.ds(row0, rows_per_tile)],
                        out_hbm.at[cid, pl.ds(row0, rows_per_tile)])

        if tail:
            @pl.when(sid == NS - 1)
            def _():
                pltpu.sync_copy(acc_sh.at[pl.ds(n_nodes - tail, tail)],
                                out_hbm.at[cid, pl.ds(n_nodes - tail, tail)])

    return agg


def _sc_edge_agg_packed(n_nodes, n_edges, chunk):
    d = 128
    T = n_nodes * 8
    e_per_w = n_edges // NW
    n_chunks = e_per_w // chunk
    share = 4992
    tail = T - share * NS
    assert share % 128 == 0 and tail % 128 == 0 and (T // 16) * 16 == T
    assert e_per_w * NW == n_edges and n_chunks * chunk == e_per_w
    assert chunk % 8 == 0 and chunk <= 128

    mesh = plsc.VectorSubcoreMesh(core_axis_name="c", subcore_axis_name="s")

    scratch = [
        pltpu.VMEM((2, chunk), jnp.int32),
        pltpu.VMEM((2, chunk), jnp.int32),
        pltpu.VMEM((2, chunk, d), jnp.float32),
        pltpu.VMEM((T,), jnp.float32),
        pltpu.SemaphoreType.DMA,
        pltpu.SemaphoreType.DMA,
        pltpu.SemaphoreType.DMA,
        pltpu.SemaphoreType.DMA,
    ]

    @functools.partial(
        pl.kernel,
        mesh=mesh,
        out_type=jax.ShapeDtypeStruct((NW * T,), jnp.float32),
        scratch_types=scratch,
        compiler_params=pltpu.CompilerParams(needs_layout_passes=False),
    )
    def agg(src_hbm, dst_hbm, vals_hbm, out_hbm,
            sbuf, dbuf, rows_v, tbl, sg0, sg1, si0, si1):
        cid = lax.axis_index("c")
        sid = lax.axis_index("s")
        wid = cid * NS + sid

        zvec = jnp.zeros((L,), jnp.float32)

        def ztbl(i, carry):
            tbl[pl.ds(i * L, L)] = zvec
            return carry

        lax.fori_loop(0, T // L, ztbl, 0)

        e_base = wid * e_per_w
        sg = (sg0, sg1, sg2)
        si = (si0, si1, si2)
        lane = lax.iota(jnp.int32, L)
        lmask = lane < 8

        def bcast(vec, u):
            idx = jnp.full((L,), u, dtype=jnp.int32)
            return lax.gather(
                vec, idx[:, None],
                dimension_numbers=lax.GatherDimensionNumbers(
                    offset_dims=(), collapsed_slice_dims=(0,),
                    start_index_map=(0,)),
                slice_sizes=(1,),
                mode=lax.GatherScatterMode.PROMISE_IN_BOUNDS)

        def idx_start(i, b):
            e0 = e_base + i * chunk
            pltpu.async_copy(src_hbm.at[pl.ds(e0, chunk)], sbuf.at[b], si[b])
            pltpu.async_copy(dst_hbm.at[pl.ds(e0, chunk)], dbuf.at[b], si[b])

        def idx_wait(b):
            pltpu.make_async_copy(src_hbm.at[pl.ds(0, chunk)], sbuf.at[b], si[b]).wait()
            pltpu.make_async_copy(dst_hbm.at[pl.ds(0, chunk)], dbuf.at[b], si[b]).wait()

        def g_start(b):
            pltpu.async_copy(vals_hbm.at[sbuf.at[b]], rows_v.at[b], sg[b])

        def g_wait(b):
            pltpu.make_async_copy(vals_hbm.at[sbuf.at[b]], rows_v.at[b], sg[b]).wait()

        idx_start(0, 0)
        idx_start(1, 1)
        idx_wait(0)
        g_start(0)

        def pair(k, carry):
            for b in (0, 1):
                i = 2 * k + b

                @pl.when(i < n_chunks)
                def _():
                    @pl.when(i + 1 < n_chunks)
                    def _():
                        idx_wait(1 - b)

                    g_wait(b)

                    @pl.when(i + 1 < n_chunks)
                    def _():
                        g_start(1 - b)

                    def ebody(g, carry2):
                        dv8 = dbuf[b, pl.ds(g * L, L)] * 8
                        for u in range(L):
                            addr = bcast(dv8, u) + lane
                            plsc.addupdate_scatter(
                                tbl, [addr], rows_v[b, g * L + u, pl.ds(0, L)],
                                mask=lmask)
                        return carry2

                    lax.fori_loop(0, chunk // L, ebody, 0)

                    @pl.when(i + 2 < n_chunks)
                    def _():
                        idx_start(i + 2, b)
            return carry

        lax.fori_loop(0, (n_chunks + 1) // 2, pair, 0)

        pltpu.sync_copy(tbl, out_hbm.at[pl.ds(wid * T, T)])

    return agg


def _mid_body(p0, p1, w1, b1, w2, out):
    agg = p0[0] + p1[0]
    h = jnp.maximum(
        jnp.dot(agg, w1[...], preferred_element_type=jnp.float32) + b1[...], 0.0)
    out[...] = jnp.dot(h, w2[...], preferred_element_type=jnp.float32)


def _fin_body(q0, q1, b2, out):
    out[...] = q0[0] + q1[0] + b2[...]


def kernel(features, edge_index, W1, b1, W2, b2):
    n, d = features.shape
    e = edge_index.shape[1]
    d2 = 128
    dn = 16
    block = 1000

    src = edge_index[0]
    dst = edge_index[1]
    W2p = jnp.pad(W2, ((0, 0), (0, d2 - W2.shape[1])))
    b1r = b1.reshape(1, d)

    p = _sc_edge_agg(n, d, e, 80, 48)(src, dst, features)

    hw2 = pl.pallas_call(
        _mid_body,
        grid=(n // block,),
        in_specs=[
            pl.BlockSpec((1, block, d), lambda i: (0, i, 0)),
            pl.BlockSpec((1, block, d), lambda i: (1, i, 0)),
            pl.BlockSpec((d, d), lambda i: (0, 0)),
            pl.BlockSpec((1, d), lambda i: (0, 0)),
            pl.BlockSpec((d, d2), lambda i: (0, 0)),
        ],
        out_specs=pl.BlockSpec((block, d2), lambda i: (i, 0)),
        out_shape=jax.ShapeDtypeStruct((n, d2), jnp.float32),
    )(p, p, W1, b1r, W2p)

    q = _sc_edge_agg(n, d2, e, 80, 48)(src, dst, hw2)

    b2p = jnp.pad(b2, (0, d2 - b2.shape[0])).reshape(1, d2)
    logits128 = pl.pallas_call(
        _fin_body,
        grid=(n // block,),
        in_specs=[
            pl.BlockSpec((1, block, d2), lambda i: (0, i, 0)),
            pl.BlockSpec((1, block, d2), lambda i: (1, i, 0)),
            pl.BlockSpec((1, d2), lambda i: (0, 0)),
        ],
        out_specs=pl.BlockSpec((block, d2), lambda i: (i, 0)),
        out_shape=jax.ShapeDtypeStruct((n, d2), jnp.float32),
    )(q, q, b2p)

    return lax.slice(logits128, (0, 0), (n, 7))


# speedup vs baseline: 1.5888x; 1.0375x over previous
import functools

import jax
import jax.numpy as jnp
from jax import lax
from jax.experimental import pallas as pl
from jax.experimental.pallas import tpu as pltpu
from jax.experimental.pallas import tpu_sc as plsc

NC = 2
NS = 16
NW = NC * NS
L = 16


def _sc_edge_agg(n_nodes, d, n_edges, chunk, zrows, dn=None):
    e_per_w = n_edges // NW
    n_chunks = e_per_w // chunk
    rows_per_tile = (n_nodes // NS) // 8 * 8
    tail = n_nodes - rows_per_tile * NS
    n_zcopy = rows_per_tile // zrows
    da = dn if dn is not None else d
    assert e_per_w * NW == n_edges and n_chunks * chunk == e_per_w
    assert n_zcopy * zrows == rows_per_tile and 0 <= tail <= zrows and tail % 8 == 0
    assert chunk % 8 == 0 and chunk <= 128 and d % L == 0 and da % L == 0

    mesh = plsc.VectorSubcoreMesh(core_axis_name="c", subcore_axis_name="s")

    scratch = [
        pltpu.VMEM((4, chunk), jnp.int32),
        pltpu.VMEM((8, chunk), jnp.int32),
        pltpu.VMEM((4, chunk, d), jnp.float32),
        pltpu.VMEM((zrows, da), jnp.float32),
        pltpu.VMEM_SHARED((n_nodes, da), jnp.float32),
        [pltpu.SemaphoreType.DMA] * 4,
        [pltpu.SemaphoreType.DMA] * 4,
        [pltpu.SemaphoreType.DMA] * 4,
    ]

    @functools.partial(
        pl.kernel,
        mesh=mesh,
        out_type=jax.ShapeDtypeStruct((NC, n_nodes, da), jnp.float32),
        scratch_types=scratch,
    )
    def agg(src_hbm, dst_hbm, vals_hbm, out_hbm,
            sbuf, dbuf, rows_v, zero_v, acc_sh, sg, si, sc):
        cid = lax.axis_index("c")
        sid = lax.axis_index("s")
        wid = cid * NS + sid
        row0 = sid * rows_per_tile

        zvec = jnp.zeros((L,), jnp.float32)
        dl = da // L

        def zstore(i, carry):
            zero_v[i // dl, pl.ds((i % dl) * L, L)] = zvec
            return carry

        lax.fori_loop(0, zrows * dl, zstore, 0)

        def zcopy(j, carry):
            pltpu.sync_copy(zero_v, acc_sh.at[pl.ds(row0 + j * zrows, zrows)])
            return carry

        lax.fori_loop(0, n_zcopy, zcopy, 0)

        if tail:
            @pl.when(sid == NS - 1)
            def _():
                pltpu.sync_copy(zero_v.at[pl.ds(0, tail)],
                                acc_sh.at[pl.ds(n_nodes - tail, tail)])

        gather_src = vals_hbm
        plsc.subcore_barrier()

        e_base = wid * e_per_w

        def idx_start(i, b, b6):
            e0 = e_base + i * chunk
            pltpu.async_copy(src_hbm.at[pl.ds(e0, chunk)], sbuf.at[b], si[b])
            pltpu.async_copy(dst_hbm.at[pl.ds(e0, chunk)], dbuf.at[b6], si[b])

        def idx_wait(b, b6):
            pltpu.make_async_copy(src_hbm.at[pl.ds(0, chunk)], sbuf.at[b], si[b]).wait()
            pltpu.make_async_copy(dst_hbm.at[pl.ds(0, chunk)], dbuf.at[b6], si[b]).wait()

        def sc_start(b, b6):
            pltpu.async_copy(rows_v.at[b], acc_sh.at[dbuf.at[b6]], sc[b], add=True)

        def sc_wait(b, b6):
            pltpu.make_async_copy(rows_v.at[b], acc_sh.at[dbuf.at[b6]], sc[b]).wait()

        def g_start(b):
            pltpu.async_copy(gather_src.at[sbuf.at[b]], rows_v.at[b], sg[b])

        def g_wait(b):
            pltpu.make_async_copy(gather_src.at[sbuf.at[b]], rows_v.at[b], sg[b]).wait()

        for j in range(4):
            idx_start(j, j, j)
        for j in range(3):
            idx_wait(j, j)
            g_start(j)

        def octo(k, carry):
            for b8 in range(8):
                i = 8 * k + b8
                b = b8 % 4
                b3i = (b + 3) % 4

                @pl.when(i < n_chunks)
                def _():
                    g_wait(b)

                    @pl.when(i >= 1)
                    def _():
                        sc_wait((b + 3) % 4, (b8 + 7) % 8)

                    sc_start(b, b8)

                    @pl.when(i + 3 < n_chunks)
                    def _():
                        idx_wait(b3i, (b8 + 3) % 8)
                        g_start(b3i)

                    @pl.when(i + 4 < n_chunks)
                    def _():
                        idx_start(i + 4, b, (b8 + 4) % 8)
            return carry

        lax.fori_loop(0, (n_chunks + 7) // 8, octo, 0)
        sc_wait((n_chunks - 1) % 4, (n_chunks - 1) % 8)
        plsc.subcore_barrier()

        pltpu.sync_copy(acc_sh.at[pl.ds(row0, rows_per_tile)],
                        out_hbm.at[cid, pl.ds(row0, rows_per_tile)])

        if tail:
            @pl.when(sid == NS - 1)
            def _():
                pltpu.sync_copy(acc_sh.at[pl.ds(n_nodes - tail, tail)],
                                out_hbm.at[cid, pl.ds(n_nodes - tail, tail)])

    return agg


def _sc_edge_agg_packed(n_nodes, n_edges, chunk):
    d = 128
    T = n_nodes * 8
    e_per_w = n_edges // NW
    n_chunks = e_per_w // chunk
    share = 4992
    tail = T - share * NS
    assert share % 128 == 0 and tail % 128 == 0 and (T // 16) * 16 == T
    assert e_per_w * NW == n_edges and n_chunks * chunk == e_per_w
    assert chunk % 8 == 0 and chunk <= 128

    mesh = plsc.VectorSubcoreMesh(core_axis_name="c", subcore_axis_name="s")

    scratch = [
        pltpu.VMEM((2, chunk), jnp.int32),
        pltpu.VMEM((2, chunk), jnp.int32),
        pltpu.VMEM((2, chunk, d), jnp.float32),
        pltpu.VMEM((T,), jnp.float32),
        pltpu.SemaphoreType.DMA,
        pltpu.SemaphoreType.DMA,
        pltpu.SemaphoreType.DMA,
        pltpu.SemaphoreType.DMA,
    ]

    @functools.partial(
        pl.kernel,
        mesh=mesh,
        out_type=jax.ShapeDtypeStruct((NW * T,), jnp.float32),
        scratch_types=scratch,
        compiler_params=pltpu.CompilerParams(needs_layout_passes=False),
    )
    def agg(src_hbm, dst_hbm, vals_hbm, out_hbm,
            sbuf, dbuf, rows_v, tbl, sg0, sg1, si0, si1):
        cid = lax.axis_index("c")
        sid = lax.axis_index("s")
        wid = cid * NS + sid

        zvec = jnp.zeros((L,), jnp.float32)

        def ztbl(i, carry):
            tbl[pl.ds(i * L, L)] = zvec
            return carry

        lax.fori_loop(0, T // L, ztbl, 0)

        e_base = wid * e_per_w
        sg = (sg0, sg1, sg2)
        si = (si0, si1, si2)
        lane = lax.iota(jnp.int32, L)
        lmask = lane < 8

        def bcast(vec, u):
            idx = jnp.full((L,), u, dtype=jnp.int32)
            return lax.gather(
                vec, idx[:, None],
                dimension_numbers=lax.GatherDimensionNumbers(
                    offset_dims=(), collapsed_slice_dims=(0,),
                    start_index_map=(0,)),
                slice_sizes=(1,),
                mode=lax.GatherScatterMode.PROMISE_IN_BOUNDS)

        def idx_start(i, b):
            e0 = e_base + i * chunk
            pltpu.async_copy(src_hbm.at[pl.ds(e0, chunk)], sbuf.at[b], si[b])
            pltpu.async_copy(dst_hbm.at[pl.ds(e0, chunk)], dbuf.at[b], si[b])

        def idx_wait(b):
            pltpu.make_async_copy(src_hbm.at[pl.ds(0, chunk)], sbuf.at[b], si[b]).wait()
            pltpu.make_async_copy(dst_hbm.at[pl.ds(0, chunk)], dbuf.at[b], si[b]).wait()

        def g_start(b):
            pltpu.async_copy(vals_hbm.at[sbuf.at[b]], rows_v.at[b], sg[b])

        def g_wait(b):
            pltpu.make_async_copy(vals_hbm.at[sbuf.at[b]], rows_v.at[b], sg[b]).wait()

        idx_start(0, 0)
        idx_start(1, 1)
        idx_wait(0)
        g_start(0)

        def pair(k, carry):
            for b in (0, 1):
                i = 2 * k + b

                @pl.when(i < n_chunks)
                def _():
                    @pl.when(i + 1 < n_chunks)
                    def _():
                        idx_wait(1 - b)

                    g_wait(b)

                    @pl.when(i + 1 < n_chunks)
                    def _():
                        g_start(1 - b)

                    def ebody(g, carry2):
                        dv8 = dbuf[b, pl.ds(g * L, L)] * 8
                        for u in range(L):
                            addr = bcast(dv8, u) + lane
                            plsc.addupdate_scatter(
                                tbl, [addr], rows_v[b, g * L + u, pl.ds(0, L)],
                                mask=lmask)
                        return carry2

                    lax.fori_loop(0, chunk // L, ebody, 0)

                    @pl.when(i + 2 < n_chunks)
                    def _():
                        idx_start(i + 2, b)
            return carry

        lax.fori_loop(0, (n_chunks + 1) // 2, pair, 0)

        pltpu.sync_copy(tbl, out_hbm.at[pl.ds(wid * T, T)])

    return agg


def _mid_body(p0, p1, w1, b1, w2, out):
    agg = p0[0] + p1[0]
    h = jnp.maximum(
        jnp.dot(agg, w1[...], preferred_element_type=jnp.float32) + b1[...], 0.0)
    out[...] = jnp.dot(h, w2[...], preferred_element_type=jnp.float32)


def _fin_body(q0, q1, b2, out):
    out[...] = q0[0] + q1[0] + b2[...]


def kernel(features, edge_index, W1, b1, W2, b2):
    n, d = features.shape
    e = edge_index.shape[1]
    d2 = 128
    dn = 16
    block = 1000

    src = edge_index[0]
    dst = edge_index[1]
    W2p = jnp.pad(W2, ((0, 0), (0, d2 - W2.shape[1])))
    b1r = b1.reshape(1, d)

    p = _sc_edge_agg(n, d, e, 80, 16)(src, dst, features)

    hw2 = pl.pallas_call(
        _mid_body,
        grid=(n // block,),
        in_specs=[
            pl.BlockSpec((1, block, d), lambda i: (0, i, 0)),
            pl.BlockSpec((1, block, d), lambda i: (1, i, 0)),
            pl.BlockSpec((d, d), lambda i: (0, 0)),
            pl.BlockSpec((1, d), lambda i: (0, 0)),
            pl.BlockSpec((d, d2), lambda i: (0, 0)),
        ],
        out_specs=pl.BlockSpec((block, d2), lambda i: (i, 0)),
        out_shape=jax.ShapeDtypeStruct((n, d2), jnp.float32),
    )(p, p, W1, b1r, W2p)

    q = _sc_edge_agg(n, d2, e, 80, 16)(src, dst, hw2)

    b2p = jnp.pad(b2, (0, d2 - b2.shape[0])).reshape(1, d2)
    logits128 = pl.pallas_call(
        _fin_body,
        grid=(n // block,),
        in_specs=[
            pl.BlockSpec((1, block, d2), lambda i: (0, i, 0)),
            pl.BlockSpec((1, block, d2), lambda i: (1, i, 0)),
            pl.BlockSpec((1, d2), lambda i: (0, 0)),
        ],
        out_specs=pl.BlockSpec((block, d2), lambda i: (i, 0)),
        out_shape=jax.ShapeDtypeStruct((n, d2), jnp.float32),
    )(q, q, b2p)

    return lax.slice(logits128, (0, 0), (n, 7))


# speedup vs baseline: 1.5898x; 1.0006x over previous
import functools

import jax
import jax.numpy as jnp
from jax import lax
from jax.experimental import pallas as pl
from jax.experimental.pallas import tpu as pltpu
from jax.experimental.pallas import tpu_sc as plsc

NC = 2
NS = 16
NW = NC * NS
L = 16


def _sc_edge_agg(n_nodes, d, n_edges, chunk, zrows):
    e_per_w = n_edges // NW
    n_chunks = e_per_w // chunk
    rows_per_tile = (n_nodes // NS) // 8 * 8
    tail = n_nodes - rows_per_tile * NS
    n_zcopy = rows_per_tile // zrows
    assert e_per_w * NW == n_edges and n_chunks * chunk == e_per_w
    assert n_zcopy * zrows == rows_per_tile and 0 <= tail <= zrows and tail % 8 == 0
    assert chunk % 8 == 0 and chunk <= 128 and d % L == 0

    mesh = plsc.VectorSubcoreMesh(core_axis_name="c", subcore_axis_name="s")

    scratch = [
        pltpu.VMEM((4, chunk), jnp.int32),
        pltpu.VMEM((8, chunk), jnp.int32),
        pltpu.VMEM((4, chunk, d), jnp.float32),
        pltpu.VMEM((zrows, d), jnp.float32),
        pltpu.VMEM_SHARED((n_nodes, d), jnp.float32),
        [pltpu.SemaphoreType.DMA] * 4,
        [pltpu.SemaphoreType.DMA] * 4,
        [pltpu.SemaphoreType.DMA] * 4,
    ]

    @functools.partial(
        pl.kernel,
        mesh=mesh,
        out_type=jax.ShapeDtypeStruct((NC, n_nodes, d), jnp.float32),
        scratch_types=scratch,
    )
    def agg(src_hbm, dst_hbm, vals_hbm, out_hbm,
            sbuf, dbuf, rows_v, zero_v, acc_sh, sg, si, sc):
        cid = lax.axis_index("c")
        sid = lax.axis_index("s")
        wid = cid * NS + sid
        row0 = sid * rows_per_tile

        zvec = jnp.zeros((L,), jnp.float32)
        dl = d // L

        def zstore(i, carry):
            zero_v[i // dl, pl.ds((i % dl) * L, L)] = zvec
            return carry

        lax.fori_loop(0, zrows * dl, zstore, 0)

        def zcopy(j, carry):
            pltpu.sync_copy(zero_v, acc_sh.at[pl.ds(row0 + j * zrows, zrows)])
            return carry

        lax.fori_loop(0, n_zcopy, zcopy, 0)

        if tail:
            @pl.when(sid == NS - 1)
            def _():
                pltpu.sync_copy(zero_v.at[pl.ds(0, tail)],
                                acc_sh.at[pl.ds(n_nodes - tail, tail)])

        plsc.subcore_barrier()

        e_base = wid * e_per_w

        def idx_start(i, b, b6):
            e0 = e_base + i * chunk
            pltpu.async_copy(src_hbm.at[pl.ds(e0, chunk)], sbuf.at[b], si[b])
            pltpu.async_copy(dst_hbm.at[pl.ds(e0, chunk)], dbuf.at[b6], si[b])

        def idx_wait(b, b6):
            pltpu.make_async_copy(src_hbm.at[pl.ds(0, chunk)], sbuf.at[b], si[b]).wait()
            pltpu.make_async_copy(dst_hbm.at[pl.ds(0, chunk)], dbuf.at[b6], si[b]).wait()

        def sc_start(b, b6):
            pltpu.async_copy(rows_v.at[b], acc_sh.at[dbuf.at[b6]], sc[b], add=True)

        def sc_wait(b, b6):
            pltpu.make_async_copy(rows_v.at[b], acc_sh.at[dbuf.at[b6]], sc[b]).wait()

        def g_start(b):
            pltpu.async_copy(vals_hbm.at[sbuf.at[b]], rows_v.at[b], sg[b])

        def g_wait(b):
            pltpu.make_async_copy(vals_hbm.at[sbuf.at[b]], rows_v.at[b], sg[b]).wait()

        for j in range(4):
            idx_start(j, j, j)
        for j in range(3):
            idx_wait(j, j)
            g_start(j)

        def octo(k, carry):
            for b8 in range(8):
                i = 8 * k + b8
                b = b8 % 4
                b3i = (b + 3) % 4

                @pl.when(i < n_chunks)
                def _():
                    g_wait(b)

                    @pl.when(i >= 1)
                    def _():
                        sc_wait((b + 3) % 4, (b8 + 7) % 8)

                    sc_start(b, b8)

                    @pl.when(i + 3 < n_chunks)
                    def _():
                        idx_wait(b3i, (b8 + 3) % 8)
                        g_start(b3i)

                    @pl.when(i + 4 < n_chunks)
                    def _():
                        idx_start(i + 4, b, (b8 + 4) % 8)
            return carry

        lax.fori_loop(0, (n_chunks + 7) // 8, octo, 0)
        sc_wait((n_chunks - 1) % 4, (n_chunks - 1) % 8)
        plsc.subcore_barrier()

        pltpu.sync_copy(acc_sh.at[pl.ds(row0, rows_per_tile)],
                        out_hbm.at[cid, pl.ds(row0, rows_per_tile)])

        if tail:
            @pl.when(sid == NS - 1)
            def _():
                pltpu.sync_copy(acc_sh.at[pl.ds(n_nodes - tail, tail)],
                                out_hbm.at[cid, pl.ds(n_nodes - tail, tail)])

    return agg


def _mid_body(p0, p1, w1, b1, w2, out):
    agg = p0[0] + p1[0]
    h = jnp.maximum(
        jnp.dot(agg, w1[...], preferred_element_type=jnp.float32) + b1[...], 0.0)
    out[...] = jnp.dot(h, w2[...], preferred_element_type=jnp.float32)


def _fin_body(q0, q1, b2, out):
    out[...] = q0[0] + q1[0] + b2[...]


def kernel(features, edge_index, W1, b1, W2, b2):
    n, d = features.shape
    e = edge_index.shape[1]
    d2 = 128
    block = 1000

    src = edge_index[0]
    dst = edge_index[1]
    W2p = jnp.pad(W2, ((0, 0), (0, d2 - W2.shape[1])))
    b1r = b1.reshape(1, d)

    p = _sc_edge_agg(n, d, e, 80, 16)(src, dst, features)

    hw2 = pl.pallas_call(
        _mid_body,
        grid=(n // block,),
        in_specs=[
            pl.BlockSpec((1, block, d), lambda i: (0, i, 0)),
            pl.BlockSpec((1, block, d), lambda i: (1, i, 0)),
            pl.BlockSpec((d, d), lambda i: (0, 0)),
            pl.BlockSpec((1, d), lambda i: (0, 0)),
            pl.BlockSpec((d, d2), lambda i: (0, 0)),
        ],
        out_specs=pl.BlockSpec((block, d2), lambda i: (i, 0)),
        out_shape=jax.ShapeDtypeStruct((n, d2), jnp.float32),
    )(p, p, W1, b1r, W2p)

    q = _sc_edge_agg(n, d2, e, 80, 16)(src, dst, hw2)

    b2p = jnp.pad(b2, (0, d2 - b2.shape[0])).reshape(1, d2)
    logits128 = pl.pallas_call(
        _fin_body,
        grid=(n // block,),
        in_specs=[
            pl.BlockSpec((1, block, d2), lambda i: (0, i, 0)),
            pl.BlockSpec((1, block, d2), lambda i: (1, i, 0)),
            pl.BlockSpec((1, d2), lambda i: (0, 0)),
        ],
        out_specs=pl.BlockSpec((block, d2), lambda i: (i, 0)),
        out_shape=jax.ShapeDtypeStruct((n, d2), jnp.float32),
    )(q, q, b2p)

    return lax.slice(logits128, (0, 0), (n, 7))
